# baseline (speedup 1.0000x reference)
import functools

import jax
import jax.numpy as jnp
from jax import lax
from jax.experimental import pallas as pl
from jax.experimental.pallas import tpu as pltpu
from jax.experimental.pallas import tpu_sc as plsc

N_FIELDS = 26
VOCAB = 100000
EMBED_DIM = 32
BATCH = 4096

_NC = 2
_NS = 16
_NW = _NC * _NS
_BPW = BATCH // _NW
_IPW = _BPW * N_FIELDS
_BC = 16
_CHUNK = _BC * N_FIELDS
_NCHUNKS = _BPW // _BC


def _sc_body(x_hbm, tab_hbm, out_hbm, xv, gv, qv, rows, outv, sem):
    wid = lax.axis_index("s") * _NC + lax.axis_index("c")
    base = wid * _IPW

    pltpu.sync_copy(x_hbm.at[pl.ds(base, _IPW)], xv)

    lane = lax.broadcasted_iota(jnp.int32, (16,), 0)

    def mk_idx(j, _):
        pos = j * 16 + lane
        f = lax.rem(pos, N_FIELDS)
        r = xv[pl.ds(j * 16, 16)] + f * VOCAB
        gv[pl.ds(j * 16, 16)] = lax.shift_right_logical(r, 2)
        qv[pl.ds(j * 16, 16)] = lax.shift_left(lax.bitwise_and(r, 3), 5)
        return 0

    lax.fori_loop(0, _IPW // 16, mk_idx, 0, unroll=False)

    lane26 = lane * N_FIELDS

    def do_chunk(c, _):
        pltpu.async_copy(
            tab_hbm.at[gv.at[pl.ds(c * _CHUNK, _CHUNK)]], rows, sem
        ).wait()

        def do_d(d, _):
            bvec = (c * _BC + lane) * EMBED_DIM + d
            acc = jnp.zeros((16,), jnp.float32)
            for f in range(N_FIELDS):
                jvec = lane26 + f
                q = plsc.load_gather(qv, [c * _CHUNK + jvec])
                acc = acc + plsc.load_gather(rows, [jvec, q + d])
            plsc.store_scatter(outv, [bvec], acc)
            return 0

        lax.fori_loop(0, EMBED_DIM, do_d, 0, unroll=False)
        return 0

    lax.fori_loop(0, _NCHUNKS, do_chunk, 0, unroll=False)

    pltpu.sync_copy(
        outv, out_hbm.at[pl.ds(wid * _BPW * EMBED_DIM, _BPW * EMBED_DIM)]
    )


@jax.jit
def _sc_embed_sum(x_flat, tab_blocks):
    mesh = plsc.VectorSubcoreMesh(core_axis_name="c", subcore_axis_name="s")
    k = functools.partial(
        pl.kernel,
        mesh=mesh,
        out_type=jax.ShapeDtypeStruct((BATCH * EMBED_DIM,), jnp.float32),
        scratch_types=[
            pltpu.VMEM((_IPW,), jnp.int32),
            pltpu.VMEM((_IPW,), jnp.int32),
            pltpu.VMEM((_IPW,), jnp.int32),
            pltpu.VMEM((_CHUNK, 128), jnp.float32),
            pltpu.VMEM((_BPW * EMBED_DIM,), jnp.float32),
            pltpu.SemaphoreType.DMA,
        ],
        compiler_params=pltpu.CompilerParams(needs_layout_passes=False),
    )(_sc_body)
    return k(x_flat, tab_blocks)


def kernel(x, tables):
    x_flat = x.astype(jnp.int32).reshape(BATCH * N_FIELDS)
    tab_blocks = tables.reshape(N_FIELDS * VOCAB * EMBED_DIM // 128, 128)
    out = _sc_embed_sum(x_flat, tab_blocks)
    return out.reshape(BATCH, EMBED_DIM)


# speedup vs baseline: 7.2940x; 7.2940x over previous
import functools

import jax
import jax.numpy as jnp
from jax import lax
from jax.experimental import pallas as pl
from jax.experimental.pallas import tpu as pltpu
from jax.experimental.pallas import tpu_sc as plsc

N_FIELDS = 26
VOCAB = 100000
EMBED_DIM = 32
BATCH = 4096

_NC = 2
_NS = 16
_H0 = 50048
_H1 = VOCAB - _H0
_NB = BATCH // 16


def _sc_body(x_hbm, tab_hbm, out_hbm, xb0, xb1, lb0, lb1, acc,
             sl0, sl1, sx0, sx1):
    wid = lax.axis_index("s") * _NC + lax.axis_index("c")
    d = wid

    zeros = jnp.zeros((16,), jnp.float32)

    def zero_acc(j, _):
        acc[pl.ds(j * 16, 16)] = zeros
        return 0

    lax.fori_loop(0, _NB, zero_acc, 0, unroll=False)

    xbufs, xsems = (xb0, xb1), (sx0, sx1)
    lbufs, lsems = (lb0, lb1), (sl0, sl1)

    def line_copy(f, h):
        if h == 0:
            src = tab_hbm.at[f, d, pl.ds(0, _H0)]
        else:
            src = tab_hbm.at[f, d, pl.ds(_H0, _H1)]
        return pltpu.async_copy(src, lbufs[h], lsems[h])

    def x_copy(f):
        return pltpu.async_copy(x_hbm.at[f, :], xbufs[f % 2], xsems[f % 2])

    xcp = x_copy(0)
    lcp = [line_copy(0, 0), line_copy(0, 1)]

    for f in range(N_FIELDS):
        xcp.wait()
        xb = xbufs[f % 2]
        if f + 1 < N_FIELDS:
            xcp = x_copy(f + 1)
        for h in range(2):
            lcp[h].wait()
            lb = lbufs[h]

            def chunk(j, _, h=h, lb=lb, xb=xb):
                v = xb[pl.ds(j * 16, 16)]
                if h == 0:
                    m = v < _H0
                    vloc = v
                else:
                    m = v >= _H0
                    vloc = v - _H0
                vloc = jnp.where(m, vloc, 0)
                val = plsc.load_gather(lb, [vloc], mask=m)
                val = jnp.where(m, val, 0.0)
                plsc.addupdate(acc.at[pl.ds(j * 16, 16)], val)
                return 0

            lax.fori_loop(0, _NB, chunk, 0, unroll=False)
            if f + 1 < N_FIELDS:
                lcp[h] = line_copy(f + 1, h)

    pltpu.sync_copy(acc, out_hbm.at[d, :])


@jax.jit
def _sc_embed_sum(x_t, tab_t):
    mesh = plsc.VectorSubcoreMesh(core_axis_name="c", subcore_axis_name="s")
    k = functools.partial(
        pl.kernel,
        mesh=mesh,
        out_type=jax.ShapeDtypeStruct((EMBED_DIM, BATCH), jnp.float32),
        scratch_types=[
            pltpu.VMEM((BATCH,), jnp.int32),
            pltpu.VMEM((BATCH,), jnp.int32),
            pltpu.VMEM((_H0,), jnp.float32),
            pltpu.VMEM((_H1,), jnp.float32),
            pltpu.VMEM((BATCH,), jnp.float32),
            pltpu.SemaphoreType.DMA,
            pltpu.SemaphoreType.DMA,
            pltpu.SemaphoreType.DMA,
            pltpu.SemaphoreType.DMA,
        ],
        compiler_params=pltpu.CompilerParams(needs_layout_passes=False),
    )(_sc_body)
    return k(x_t, tab_t)


def kernel(x, tables):
    x_t = x.astype(jnp.int32).T
    tab_t = tables.transpose(0, 2, 1)
    out_t = _sc_embed_sum(x_t, tab_t)
    return out_t.T


# speedup vs baseline: 8.2976x; 1.1376x over previous
import functools

import jax
import jax.numpy as jnp
from jax import lax
from jax.experimental import pallas as pl
from jax.experimental.pallas import tpu as pltpu
from jax.experimental.pallas import tpu_sc as plsc

N_FIELDS = 26
VOCAB = 100000
EMBED_DIM = 32
BATCH = 4096

_NC = 2
_NS = 16
_T0 = 33408
_TSTARTS = (0, _T0, 2 * _T0)
_TSIZES = (_T0, _T0, VOCAB - 2 * _T0)
_NB = BATCH // 16


def _sc_body(x_hbm, tab_hbm, out_hbm, xb0, xb1, lb0, lb1, lb2, acc,
             sl0, sl1, sl2, sx0, sx1):
    wid = lax.axis_index("s") * _NC + lax.axis_index("c")
    d = wid

    zeros = jnp.zeros((16,), jnp.float32)

    def zero_acc(j, _):
        acc[pl.ds(j * 16, 16)] = zeros
        return 0

    lax.fori_loop(0, _NB, zero_acc, 0, unroll=False)

    xbufs, xsems = (xb0, xb1), (sx0, sx1)
    lbufs, lsems = (lb0, lb1, lb2), (sl0, sl1, sl2)

    def line_copy(f, h):
        src = tab_hbm.at[f, d, pl.ds(_TSTARTS[h], _TSIZES[h])]
        return pltpu.async_copy(src, lbufs[h], lsems[h])

    def x_copy(f):
        return pltpu.async_copy(x_hbm.at[f, :], xbufs[f % 2], xsems[f % 2])

    xcp = x_copy(0)
    lcp = [line_copy(0, 0), line_copy(0, 1), line_copy(0, 2)]

    for f in range(N_FIELDS):
        xcp.wait()
        xb = xbufs[f % 2]
        if f + 1 < N_FIELDS:
            xcp = x_copy(f + 1)
        for h in range(3):
            lcp[h].wait()
            lb = lbufs[h]

            def chunk(j, _, h=h, lb=lb, xb=xb):
                v = xb[pl.ds(j * 16, 16)]
                lo, sz = _TSTARTS[h], _TSIZES[h]
                vloc = v - lo
                if h == 0:
                    m = v < sz
                elif h == 2:
                    m = v >= lo
                else:
                    m = jnp.logical_and(v >= lo, v < lo + sz)
                vloc = jnp.where(m, vloc, 0)
                val = plsc.load_gather(lb, [vloc], mask=m)
                val = jnp.where(m, val, 0.0)
                plsc.addupdate(acc.at[pl.ds(j * 16, 16)], val)
                return 0

            lax.fori_loop(0, _NB, chunk, 0, unroll=False)
            if f + 1 < N_FIELDS:
                lcp[h] = line_copy(f + 1, h)

    pltpu.sync_copy(acc, out_hbm.at[d, :])


@jax.jit
def _sc_embed_sum(x_t, tab_t):
    mesh = plsc.VectorSubcoreMesh(core_axis_name="c", subcore_axis_name="s")
    k = functools.partial(
        pl.kernel,
        mesh=mesh,
        out_type=jax.ShapeDtypeStruct((EMBED_DIM, BATCH), jnp.float32),
        scratch_types=[
            pltpu.VMEM((BATCH,), jnp.int32),
            pltpu.VMEM((BATCH,), jnp.int32),
            pltpu.VMEM((_TSIZES[0],), jnp.float32),
            pltpu.VMEM((_TSIZES[1],), jnp.float32),
            pltpu.VMEM((_TSIZES[2],), jnp.float32),
            pltpu.VMEM((BATCH,), jnp.float32),
            pltpu.SemaphoreType.DMA,
            pltpu.SemaphoreType.DMA,
            pltpu.SemaphoreType.DMA,
            pltpu.SemaphoreType.DMA,
            pltpu.SemaphoreType.DMA,
        ],
        compiler_params=pltpu.CompilerParams(needs_layout_passes=False),
    )(_sc_body)
    return k(x_t, tab_t)


def kernel(x, tables):
    x_t = x.astype(jnp.int32).T
    tab_t = tables.transpose(0, 2, 1)
    out_t = _sc_embed_sum(x_t, tab_t)
    return out_t.T
